# Initial kernel scaffold; baseline (speedup 1.0000x reference)
import jax
import jax.numpy as jnp
from jax.experimental import pallas as pl


def kernel(rel_idx, position_idx, rel_table, pos_table):
    raise NotImplementedError("write your pallas kernel here")


# speedup vs baseline: 5.7412x; 5.7412x over previous
import functools

import jax
import jax.numpy as jnp
from jax import lax
from jax.experimental import pallas as pl
from jax.experimental.pallas import tpu as pltpu
from jax.experimental.pallas import tpu_sc as plsc

_D = 32
_CHUNK = 128


@functools.cache
def _build(n_total, rel_vocab, pos_vocab):
    info = plsc.get_sparse_core_info()
    nw = info.num_cores * info.num_subcores
    per_w = n_total // nw
    assert n_total % nw == 0 and per_w % _CHUNK == 0
    n_chunks = per_w // _CHUNK

    mesh = plsc.VectorSubcoreMesh(core_axis_name="c", subcore_axis_name="s")

    @functools.partial(
        pl.kernel,
        mesh=mesh,
        compiler_params=pltpu.CompilerParams(use_tc_tiling_on_sc=False),
        out_type=jax.ShapeDtypeStruct((n_total, 2 * _D), jnp.float32),
        scratch_types=[
            pltpu.VMEM((_CHUNK,), jnp.int32),
            pltpu.VMEM((_CHUNK,), jnp.int32),
            pltpu.VMEM((_CHUNK, _D), jnp.float32),
            pltpu.VMEM((_CHUNK, _D), jnp.float32),
            pltpu.SemaphoreType.DMA,
        ],
    )
    def sc_kernel(rel_idx_hbm, pos_idx_hbm, rel_tab_hbm, pos_tab_hbm, out_hbm,
                  rel_iv, pos_iv, rel_rows, pos_rows, sem):
        wid = lax.axis_index("s") * info.num_cores + lax.axis_index("c")
        base0 = wid * per_w

        def step(i, carry):
            base = base0 + i * _CHUNK
            pltpu.sync_copy(rel_idx_hbm.at[pl.ds(base, _CHUNK)], rel_iv)
            pltpu.sync_copy(pos_idx_hbm.at[pl.ds(base, _CHUNK)], pos_iv)
            pltpu.async_copy(rel_tab_hbm.at[rel_iv], rel_rows, sem).wait()
            pltpu.async_copy(pos_tab_hbm.at[pos_iv], pos_rows, sem).wait()
            pltpu.sync_copy(rel_rows, out_hbm.at[pl.ds(base, _CHUNK), pl.ds(0, _D)])
            pltpu.sync_copy(pos_rows, out_hbm.at[pl.ds(base, _CHUNK), pl.ds(_D, _D)])
            return carry

        lax.fori_loop(0, n_chunks, step, 0)

    return sc_kernel


def kernel(rel_idx, position_idx, rel_table, pos_table):
    b, l = rel_idx.shape
    n = b * l
    rel_flat = rel_idx.reshape(n).astype(jnp.int32)
    pos_flat = position_idx.reshape(n).astype(jnp.int32)
    sc = _build(n, rel_table.shape[0], pos_table.shape[0])
    out = sc(rel_flat, pos_flat, rel_table, pos_table)
    return out.reshape(b, l, 2 * _D)


# speedup vs baseline: 8.3704x; 1.4580x over previous
import functools

import jax
import jax.numpy as jnp
from jax import lax
from jax.experimental import pallas as pl
from jax.experimental.pallas import tpu as pltpu
from jax.experimental.pallas import tpu_sc as plsc

_D = 32
_G = 128
_C = 256
_NBUF = 4
_BLK = _C * _NBUF


@functools.cache
def _build(n_total):
    info = plsc.get_sparse_core_info()
    nw = info.num_cores * info.num_subcores
    per_w = n_total // nw
    assert n_total % nw == 0 and per_w % _BLK == 0
    n_blk = per_w // _BLK
    n_g = _C // _G

    mesh = plsc.VectorSubcoreMesh(core_axis_name="c", subcore_axis_name="s")

    @functools.partial(
        pl.kernel,
        mesh=mesh,
        compiler_params=pltpu.CompilerParams(use_tc_tiling_on_sc=False),
        out_type=jax.ShapeDtypeStruct((n_total, 2 * _D), jnp.float32),
        scratch_types=[
            pltpu.VMEM((2, _BLK), jnp.int32),
            pltpu.VMEM((2, _BLK), jnp.int32),
            pltpu.VMEM((_NBUF, _C, _D), jnp.float32),
            pltpu.VMEM((_NBUF, _C, _D), jnp.float32),
            pltpu.SemaphoreType.DMA((2,)),
            pltpu.SemaphoreType.DMA((_NBUF,)),
            pltpu.SemaphoreType.DMA((_NBUF,)),
        ],
    )
    def sc_kernel(rel_idx_hbm, pos_idx_hbm, rel_tab_hbm, pos_tab_hbm, out_hbm,
                  rel_iv, pos_iv, rel_rows, pos_rows, isem, gsem, wsem):
        wid = lax.axis_index("s") * info.num_cores + lax.axis_index("c")
        base0 = wid * per_w

        def load_idx(blk, slot):
            b = base0 + blk * _BLK
            pltpu.async_copy(rel_idx_hbm.at[pl.ds(b, _BLK)], rel_iv.at[slot],
                             isem.at[slot])
            pltpu.async_copy(pos_idx_hbm.at[pl.ds(b, _BLK)], pos_iv.at[slot],
                             isem.at[slot])

        def wait_idx(slot):
            pltpu.make_async_copy(rel_idx_hbm.at[pl.ds(0, _BLK)],
                                  rel_iv.at[slot], isem.at[slot]).wait()
            pltpu.make_async_copy(pos_idx_hbm.at[pl.ds(0, _BLK)],
                                  pos_iv.at[slot], isem.at[slot]).wait()

        def issue_gathers(islot, b):
            cps = []
            for j in range(n_g):
                off = b * _C + j * _G
                cps.append(pltpu.async_copy(
                    rel_tab_hbm.at[rel_iv.at[islot, pl.ds(off, _G)]],
                    rel_rows.at[b, pl.ds(j * _G, _G), :], gsem.at[b]))
                cps.append(pltpu.async_copy(
                    pos_tab_hbm.at[pos_iv.at[islot, pl.ds(off, _G)]],
                    pos_rows.at[b, pl.ds(j * _G, _G), :], gsem.at[b]))
            return cps

        def issue_writes(blk, b):
            gb = base0 + blk * _BLK + b * _C
            return [
                pltpu.async_copy(rel_rows.at[b],
                                 out_hbm.at[pl.ds(gb, _C), pl.ds(0, _D)],
                                 wsem.at[b]),
                pltpu.async_copy(pos_rows.at[b],
                                 out_hbm.at[pl.ds(gb, _C), pl.ds(_D, _D)],
                                 wsem.at[b]),
            ]

        def wait_writes(b):
            for col in (0, _D):
                pltpu.make_async_copy(
                    rel_rows.at[b],
                    out_hbm.at[pl.ds(0, _C), pl.ds(col, _D)],
                    wsem.at[b]).wait()

        load_idx(0, 0)
        load_idx(1, 1)
        wait_idx(0)
        gcps = [issue_gathers(0, b) for b in range(_NBUF)]
        for b in range(_NBUF):
            for cp in gcps[b]:
                cp.wait()
            issue_writes(0, b)

        def body(k, carry):
            sk = lax.rem(k, 2)
            sn = 1 - sk
            nxt = jnp.minimum(k + 1, n_blk - 1)
            load_idx(nxt, sn)
            wait_idx(sk)
            gcps = []
            for b in range(_NBUF):
                wait_writes(b)
                gcps.append(issue_gathers(sk, b))
            for b in range(_NBUF):
                for cp in gcps[b]:
                    cp.wait()
                issue_writes(k, b)
            return carry

        lax.fori_loop(1, n_blk, body, 0)
        wait_idx(n_blk % 2)
        for b in range(_NBUF):
            wait_writes(b)

    return sc_kernel


def kernel(rel_idx, position_idx, rel_table, pos_table):
    b, l = rel_idx.shape
    n = b * l
    rel_flat = rel_idx.reshape(n).astype(jnp.int32)
    pos_flat = position_idx.reshape(n).astype(jnp.int32)
    out = _build(n)(rel_flat, pos_flat, rel_table, pos_table)
    return out.reshape(b, l, 2 * _D)
